# baseline (speedup 1.0000x reference)
import functools

import jax
import jax.numpy as jnp
from jax import lax
from jax.experimental import pallas as pl
from jax.experimental.pallas import tpu as pltpu
from jax.experimental.pallas import tpu_sc as plsc


_LANES = 16
_SC = 8


def _make_kernel(B, S, D, V):
    info = plsc.get_sparse_core_info()
    NC, NS = info.num_cores, info.num_subcores
    NW = NC * NS
    BT = B // 128
    assert BT == NW and B % 128 == 0
    assert S % _SC == 0
    n_chunks = S // _SC
    DT = D // 8
    assert DT * 8 == D and D % _LANES == 0
    NV = 128 // _LANES

    mesh = plsc.VectorSubcoreMesh(core_axis_name="c", subcore_axis_name="s")

    @functools.partial(
        pl.kernel,
        mesh=mesh,
        compiler_params=pltpu.CompilerParams(
            use_tc_tiling_on_sc=False, needs_layout_passes=False
        ),
        out_type=jax.ShapeDtypeStruct((S, DT, BT, 8, 128), jnp.float32),
        scratch_types=[
            pltpu.VMEM((_SC, 128), jnp.int32),
            pltpu.VMEM((_SC, 128, D + 1), jnp.float32),
            pltpu.VMEM((_SC, DT, 1, 8, 128), jnp.float32),
            pltpu.VMEM((S, D), jnp.float32),
            pltpu.SemaphoreType.DMA,
        ],
    )
    def k(xt_hbm, item_hbm, pos_hbm, out_hbm, idx_v, rbuf33, obuf,
          pos_v, sem):
        wid = lax.axis_index("s") * NC + lax.axis_index("c")
        b0 = wid * 128

        pltpu.sync_copy(pos_hbm, pos_v)

        lane = lax.iota(jnp.int32, _LANES)
        cis = [lane + _LANES * kk for kk in range(NV)]

        def chunk_body(i, carry):
            s0 = i * _SC
            pltpu.sync_copy(
                xt_hbm.at[pl.ds(s0, _SC), pl.ds(b0, 128)], idx_v
            )

            copies = []
            for si in range(_SC):
                copies.append(
                    pltpu.async_copy(
                        item_hbm.at[idx_v.at[si]], rbuf33.at[si], sem
                    )
                )
            for c in copies:
                c.wait()

            def pos_body(si, c1):
                c_si = jnp.full((_LANES,), si, jnp.int32)
                c_s = jnp.full((_LANES,), s0 + si, jnp.int32)
                for d in range(D):
                    c_d = jnp.full((_LANES,), d, jnp.int32)
                    pv = plsc.load_gather(pos_v, [c_s, c_d])
                    tr = d // 8
                    r = d % 8
                    for kk in range(NV):
                        v = plsc.load_gather(rbuf33, [c_si, cis[kk], c_d])
                        obuf[si, tr, 0, r, pl.ds(kk * _LANES, _LANES)] = v + pv
                return c1

            lax.fori_loop(0, _SC, pos_body, 0)

            pltpu.sync_copy(
                obuf, out_hbm.at[pl.ds(s0, _SC), :, pl.ds(wid, 1)]
            )
            return carry

        lax.fori_loop(0, n_chunks, chunk_body, 0)

    return k


def kernel(x, item_emb, pos_emb):
    B, S = x.shape
    V, D = item_emb.shape
    xt = x.astype(jnp.int32).T
    item33 = jnp.pad(item_emb, ((0, 0), (0, 1)))
    out5 = _make_kernel(B, S, D, V)(xt, item33, pos_emb[:S])
    return jnp.transpose(out5, (2, 4, 0, 1, 3)).reshape(B, S, D)


# speedup vs baseline: 1.2928x; 1.2928x over previous
import functools

import jax
import jax.numpy as jnp
from jax import lax
from jax.experimental import pallas as pl
from jax.experimental.pallas import tpu as pltpu
from jax.experimental.pallas import tpu_sc as plsc


_LANES = 16
_SC = 8


def _make_kernel(B, S, D, V):
    info = plsc.get_sparse_core_info()
    NC, NS = info.num_cores, info.num_subcores
    NW = NC * NS
    BT = B // 128
    assert BT == NW and B % 128 == 0
    assert S % _SC == 0
    n_chunks = S // _SC
    HREG = D // _LANES
    assert HREG * _LANES == D

    mesh = plsc.VectorSubcoreMesh(core_axis_name="c", subcore_axis_name="s")

    @functools.partial(
        pl.kernel,
        mesh=mesh,
        compiler_params=pltpu.CompilerParams(
            use_tc_tiling_on_sc=False, needs_layout_passes=False
        ),
        out_type=jax.ShapeDtypeStruct((S, B, D), jnp.float32),
        scratch_types=[
            pltpu.VMEM((_SC, 128), jnp.int32),
            pltpu.VMEM((_SC, 128, D), jnp.float32),
            pltpu.VMEM((S, D), jnp.float32),
            pltpu.SemaphoreType.DMA,
        ],
    )
    def k(xt_hbm, item_hbm, pos_hbm, out_hbm, idx_v, rbuf, pos_v, sem):
        wid = lax.axis_index("s") * NC + lax.axis_index("c")
        b0 = wid * 128

        pltpu.sync_copy(pos_hbm, pos_v)

        def chunk_body(i, carry):
            s0 = i * _SC
            pltpu.sync_copy(
                xt_hbm.at[pl.ds(s0, _SC), pl.ds(b0, 128)], idx_v
            )

            copies = []
            for si in range(_SC):
                copies.append(
                    pltpu.async_copy(
                        item_hbm.at[idx_v.at[si]], rbuf.at[si], sem
                    )
                )
            for c in copies:
                c.wait()

            def pos_body(si, c1):
                pvs = [
                    pos_v[s0 + si, pl.ds(h * _LANES, _LANES)]
                    for h in range(HREG)
                ]

                def row_body(c, c2):
                    for h in range(HREG):
                        sl = pl.ds(h * _LANES, _LANES)
                        rbuf[si, c, sl] = rbuf[si, c, sl] + pvs[h]
                    return c2

                lax.fori_loop(0, 128, row_body, 0)
                return c1

            lax.fori_loop(0, _SC, pos_body, 0)

            pltpu.sync_copy(
                rbuf, out_hbm.at[pl.ds(s0, _SC), pl.ds(b0, 128)]
            )
            return carry

        lax.fori_loop(0, n_chunks, chunk_body, 0)

    return k


def kernel(x, item_emb, pos_emb):
    B, S = x.shape
    V, D = item_emb.shape
    xt = x.astype(jnp.int32).T
    out_sbd = _make_kernel(B, S, D, V)(xt, item_emb, pos_emb[:S])
    return jnp.transpose(out_sbd, (1, 0, 2))
